# baseline (speedup 1.0000x reference)
import functools
import math

import jax
import jax.numpy as jnp
from jax import lax
from jax.experimental import pallas as pl
from jax.experimental.pallas import tpu as pltpu
from jax.experimental.pallas import tpu_sc as plsc

D_MODEL = 64
N_LOOKUPS = 16384 * 50
SCALE = math.sqrt(D_MODEL)

_info = plsc.get_sparse_core_info()
_NC, _NS = _info.num_cores, _info.num_subcores
_NW = _NC * _NS
_PER_W = N_LOOKUPS // _NW

CHUNK = 512
NIDX = CHUNK // 128
N_CHUNKS = _PER_W // CHUNK
IDX_ROWS = N_LOOKUPS // 128


def _emb_body(x_hbm, lut_hbm, out_hbm, idx_v, rows_v, sem):
    wid = lax.axis_index("s") * _NC + lax.axis_index("c")
    row_base = wid * (_PER_W // 128)

    def chunk(g, carry):
        r0 = row_base + g * NIDX
        pltpu.sync_copy(x_hbm.at[pl.ds(r0, NIDX)], idx_v)
        copies = [
            pltpu.async_copy(lut_hbm.at[idx_v.at[j]], rows_v.at[j], sem)
            for j in range(NIDX)
        ]
        for cp in copies:
            cp.wait()

        def scale_row(r, c2):
            for j in range(NIDX):
                for k in range(D_MODEL // 16):
                    sl = pl.ds(k * 16, 16)
                    rows_v[j, r, sl] = rows_v[j, r, sl] * SCALE
            return c2

        lax.fori_loop(0, 128, scale_row, 0)
        pltpu.sync_copy(rows_v, out_hbm.at[pl.ds(r0, NIDX)])
        return carry

    lax.fori_loop(0, N_CHUNKS, chunk, 0)


_emb = functools.partial(
    pl.kernel,
    out_type=jax.ShapeDtypeStruct((IDX_ROWS, 128, D_MODEL), jnp.float32),
    mesh=plsc.VectorSubcoreMesh(core_axis_name="c", subcore_axis_name="s"),
    scratch_types=[
        pltpu.VMEM((NIDX, 128), jnp.int32),
        pltpu.VMEM((NIDX, 128, D_MODEL), jnp.float32),
        pltpu.SemaphoreType.DMA,
    ],
    compiler_params=pltpu.CompilerParams(use_tc_tiling_on_sc=False),
)(_emb_body)


@jax.jit
def kernel(x, lut):
    x2 = x.reshape(IDX_ROWS, 128).astype(jnp.int32)
    out = _emb(x2, lut)
    return out.reshape(16384, 50, D_MODEL)


# speedup vs baseline: 1.0889x; 1.0889x over previous
import functools
import math

import jax
import jax.numpy as jnp
from jax import lax
from jax.experimental import pallas as pl
from jax.experimental.pallas import tpu as pltpu
from jax.experimental.pallas import tpu_sc as plsc

D_MODEL = 64
N_LOOKUPS = 16384 * 50
SCALE = math.sqrt(D_MODEL)

_info = plsc.get_sparse_core_info()
_NC, _NS = _info.num_cores, _info.num_subcores
_NW = _NC * _NS
_PER_W = N_LOOKUPS // _NW

CHUNK = 256
NIDX = CHUNK // 128
N_CHUNKS = _PER_W // CHUNK
NBUF = 4
N_OUTER = N_CHUNKS // NBUF
IDX_ROWS = N_LOOKUPS // 128
W_ROWS = _PER_W // 128


def _emb_body(x_hbm, lut_hbm, out_hbm, idx_all, rows, sems_g, sems_o):
    wid = lax.axis_index("s") * _NC + lax.axis_index("c")
    row_base = wid * W_ROWS

    def gather_start(g, b):
        for j in range(NIDX):
            pltpu.async_copy(
                lut_hbm.at[idx_all.at[g * NIDX + j]], rows[b].at[j], sems_g[b]
            )

    def gather_wait(g, b):
        for j in range(NIDX):
            pltpu.make_async_copy(
                lut_hbm.at[idx_all.at[g * NIDX + j]], rows[b].at[j], sems_g[b]
            ).wait()

    def out_start(g, b):
        pltpu.async_copy(
            rows[b], out_hbm.at[pl.ds(row_base + g * NIDX, NIDX)], sems_o[b]
        )

    def out_wait(g, b):
        pltpu.make_async_copy(
            rows[b], out_hbm.at[pl.ds(row_base + g * NIDX, NIDX)], sems_o[b]
        ).wait()

    def scale(b):
        def srow(r, c):
            for j in range(NIDX):
                for k in range(D_MODEL // 16):
                    sl = pl.ds(k * 16, 16)
                    rows[b][j, r, sl] = rows[b][j, r, sl] * SCALE
            return c

        lax.fori_loop(0, 128, srow, 0)

    pltpu.sync_copy(x_hbm.at[pl.ds(row_base, W_ROWS)], idx_all)

    gather_start(0, 0)
    gather_start(1, 1)

    def step(g, b):
        gather_wait(g, b)
        scale(b)
        out_start(g, b)

    for b in range(NBUF):
        g = b
        step(g, b)
        if g >= 2:
            out_wait(g - 2, (g + 2) % NBUF)
        gather_start(g + 2, (g + 2) % NBUF)

    def outer(t, c):
        for b in range(NBUF):
            g = NBUF * t + b
            b2 = (b + 2) % NBUF
            step(g, b)
            out_wait(g - 2, b2)
            gather_start(g + 2, b2)
        return c

    lax.fori_loop(1, N_OUTER - 1, outer, 0)

    for b in range(NBUF):
        g = N_CHUNKS - NBUF + b
        step(g, b)
        out_wait(g - 2, (g + 2) % NBUF)
        if g + 2 < N_CHUNKS:
            gather_start(g + 2, (g + 2) % NBUF)

    out_wait(N_CHUNKS - 2, (N_CHUNKS - 2) % NBUF)
    out_wait(N_CHUNKS - 1, (N_CHUNKS - 1) % NBUF)


_emb = functools.partial(
    pl.kernel,
    out_type=jax.ShapeDtypeStruct((IDX_ROWS, 128, D_MODEL), jnp.float32),
    mesh=plsc.VectorSubcoreMesh(core_axis_name="c", subcore_axis_name="s"),
    scratch_types=[
        pltpu.VMEM((W_ROWS, 128), jnp.int32),
        [pltpu.VMEM((NIDX, 128, D_MODEL), jnp.float32) for _ in range(NBUF)],
        [pltpu.SemaphoreType.DMA for _ in range(NBUF)],
        [pltpu.SemaphoreType.DMA for _ in range(NBUF)],
    ],
    compiler_params=pltpu.CompilerParams(use_tc_tiling_on_sc=False),
)(_emb_body)


@jax.jit
def kernel(x, lut):
    x2 = x.reshape(IDX_ROWS, 128).astype(jnp.int32)
    out = _emb(x2, lut)
    return out.reshape(16384, 50, D_MODEL)
